# Initial kernel scaffold; baseline (speedup 1.0000x reference)
import jax
import jax.numpy as jnp
from jax.experimental import pallas as pl


def kernel(h_bond, bond_index, h_node, bond_time, L_Wb, L_Wn, L_W1, L_b1, L_W2, L_b2, L_Wg1, L_bg1, L_Wg2, L_bg2, R_Wb, R_Wn, R_W1, R_b1, R_W2, R_b2, R_Wg1, R_bg1, R_Wg2, R_bg2, nl_W, nl_b, nr_W, nr_b, sf_W, sf_b, ln_g, ln_b, ot_W, ot_b):
    raise NotImplementedError("write your pallas kernel here")


# speedup vs baseline: 2.6316x; 2.6316x over previous
import jax
import jax.numpy as jnp
from jax import lax
from jax.experimental import pallas as pl
from jax.experimental.pallas import tpu as pltpu
from jax.experimental.pallas import tpu_sc as plsc

_N = 10000
_E = 160000
_D = 128
_DI = 256
_DG = 32
_NC, _NS = 2, 16
_NW = _NC * _NS
_CH = 80
_U = 5
_ROWS_W = (2 * _E) // _NW
_NCHUNK = _ROWS_W // _CH
_NSUP = _NCHUNK // _U
_NP = 10240
_RPT = _NP // _NS
_CHS = 40
_NCHS = (_E // _NS) // _CHS
_NSUPS = _NCHS // _U
_EPT = _E // _NS
_BE = 1000
_GRID = _E // _BE


def _sc_mesh():
    return plsc.VectorSubcoreMesh(
        core_axis_name="c", subcore_axis_name="s",
        num_cores=_NC, num_subcores=_NS)


def _gather_rows(table, idx_flat, add_half_offset):
    t_rows, w = table.shape
    m = idx_flat.shape[0]
    dt = table.dtype

    def body(table_hbm, idx_hbm, out_hbm, idxv, buf, gsem, ssem):
        c = lax.axis_index("c")
        s = lax.axis_index("s")
        wid = s * _NC + c
        rbase = pl.multiple_of(wid * _ROWS_W, _ROWS_W)
        pltpu.sync_copy(idx_hbm.at[pl.ds(rbase, _ROWS_W)], idxv)
        if add_half_offset:
            off = jnp.where(wid >= _NW // 2,
                            jnp.int32(t_rows // 2), jnp.int32(0))
            offv = jnp.zeros((16,), jnp.int32) + off

            def adj(k, carry):
                b = pl.multiple_of(k * 16, 16)
                idxv[pl.ds(b, 16)] = idxv[pl.ds(b, 16)] + offv
                return carry

            lax.fori_loop(0, _ROWS_W // 16, adj, 0, unroll=False)

        def sup(t, carry):
            g0 = t * _U
            gets = [
                pltpu.async_copy(
                    table_hbm.at[idxv.at[pl.ds(
                        pl.multiple_of((g0 + b) * _CH, _CH), _CH)]],
                    buf.at[b], gsem)
                for b in range(_U)]
            for d in gets:
                d.wait()
            puts = [
                pltpu.async_copy(
                    buf.at[b],
                    out_hbm.at[pl.ds(rbase + (g0 + b) * _CH, _CH)],
                    ssem)
                for b in range(_U)]
            for d in puts:
                d.wait()
            return carry

        lax.fori_loop(0, _NSUP, sup, 0, unroll=False)

    f = pl.kernel(
        body,
        out_type=jax.ShapeDtypeStruct((m, w), dt),
        mesh=_sc_mesh(),
        scratch_types=[
            pltpu.VMEM((_ROWS_W,), jnp.int32),
            pltpu.VMEM((_U, _CH, w), dt),
            pltpu.SemaphoreType.DMA,
            pltpu.SemaphoreType.DMA,
        ])
    return f(table, idx_flat)


def _scatter_sum(msgs, idx_flat, zeros):
    def body(msgs_hbm, idx_hbm, zr_hbm, seg_hbm, idxv, buf, shared,
             gsem, ssem):
        c = lax.axis_index("c")
        s = lax.axis_index("s")
        zb = pl.multiple_of(s * _RPT, _RPT)
        pltpu.sync_copy(zr_hbm.at[pl.ds(zb, _RPT)], shared.at[pl.ds(zb, _RPT)])
        pltpu.sync_copy(
            idx_hbm.at[pl.ds(pl.multiple_of(c * _E + s * _EPT, _EPT), _EPT)],
            idxv)
        plsc.subcore_barrier()
        ebase = pl.multiple_of(s * _EPT, _EPT)

        def sup(t, carry):
            g0 = t * _U
            gets = [
                pltpu.async_copy(
                    msgs_hbm.at[c, pl.ds(ebase + (g0 + b) * _CHS, _CHS)],
                    buf.at[b], gsem)
                for b in range(_U)]
            for d in gets:
                d.wait()
            puts = [
                pltpu.async_copy(
                    buf.at[b],
                    shared.at[idxv.at[pl.ds(
                        pl.multiple_of((g0 + b) * _CHS, _CHS), _CHS)]],
                    ssem, add=True)
                for b in range(_U)]
            for d in puts:
                d.wait()
            return carry

        lax.fori_loop(0, _NSUPS, sup, 0, unroll=False)
        plsc.subcore_barrier()
        pltpu.sync_copy(shared.at[pl.ds(zb, _RPT)],
                        seg_hbm.at[pl.ds(c * _NP + zb, _RPT)])

    f = pl.kernel(
        body,
        out_type=jax.ShapeDtypeStruct((2 * _NP, _D), jnp.float32),
        mesh=_sc_mesh(),
        scratch_types=[
            pltpu.VMEM((_EPT,), jnp.int32),
            pltpu.VMEM((_U, _CHS, _D), jnp.float32),
            pltpu.VMEM_SHARED((_NP, _D), jnp.float32),
            pltpu.SemaphoreType.DMA,
            pltpu.SemaphoreType.DMA,
        ])
    return f(msgs, idx_flat, zeros)


def _tc1_body(hb_ref, hl_ref, hr_ref, gtl_ref, gtr_ref,
              lWb, lWn, lW1, lb1, lW2, lb2, lG1b, lG1n, lWg2, lbg2,
              rWb, rWn, rW1, rb1, rW2, rb2, rG1b, rG1n, rWg2, rbg2,
              nlW, nrW, sfW, lb_ref,
              msgs_ref, lin_ref):
    f32 = jnp.float32
    bf16 = jnp.bfloat16
    hbb = hb_ref[...].astype(bf16)
    hl = hl_ref[...].astype(bf16)
    hr = hr_ref[...].astype(bf16)

    def side(hn, Wb, Wn, W1, b1, W2, b2, G1b, G1n, gt, Wg2, bg2):
        a = jnp.dot(hbb, Wb[...], preferred_element_type=f32)
        bb = jnp.dot(hn, Wn[...], preferred_element_type=f32)
        inter = (a * bb).astype(bf16)
        h1 = jax.nn.relu(
            jnp.dot(inter, W1[...], preferred_element_type=f32) + b1[...])
        inter2 = (jnp.dot(h1.astype(bf16), W2[...],
                          preferred_element_type=f32) + b2[...])
        g = (jnp.dot(hbb, G1b[...], preferred_element_type=f32)
             + jnp.dot(hn, G1n[...], preferred_element_type=f32)
             + gt[...].astype(f32))
        g = jax.nn.relu(g).astype(bf16)
        gate = jnp.dot(g, Wg2[...], preferred_element_type=f32) + bg2[...]
        return inter2 * jax.nn.sigmoid(gate)

    msgs_ref[0] = side(hl, lWb, lWn, lW1, lb1, lW2, lb2,
                       lG1b, lG1n, gtl_ref, lWg2, lbg2)
    msgs_ref[1] = side(hr, rWb, rWn, rW1, rb1, rW2, rb2,
                       rG1b, rG1n, gtr_ref, rWg2, rbg2)
    lin = (jnp.dot(hl, nlW[...], preferred_element_type=f32)
           + jnp.dot(hr, nrW[...], preferred_element_type=f32)
           + jnp.dot(hbb, sfW[...], preferred_element_type=f32)
           + lb_ref[...])
    lin_ref[...] = lin.astype(bf16)


def _tc2_body(gl_ref, gr_ref, lin_ref, lng, lnb, otW, otb, out_ref):
    f32 = jnp.float32
    h = (gl_ref[...].astype(f32) + gr_ref[...].astype(f32)
         + lin_ref[...].astype(f32))
    mu = jnp.mean(h, axis=1, keepdims=True)
    d = h - mu
    var = jnp.mean(d * d, axis=1, keepdims=True)
    hn = d * lax.rsqrt(var + 1e-5) * lng[...] + lnb[...]
    r = jax.nn.relu(hn).astype(jnp.bfloat16)
    out_ref[...] = jnp.dot(r, otW[...], preferred_element_type=f32) + otb[...]


def _full_spec(shape):
    return pl.BlockSpec(shape, lambda i: tuple(0 for _ in shape))


def kernel(h_bond, bond_index, h_node, bond_time,
           L_Wb, L_Wn, L_W1, L_b1, L_W2, L_b2, L_Wg1, L_bg1, L_Wg2, L_bg2,
           R_Wb, R_Wn, R_W1, R_b1, R_W2, R_b2, R_Wg1, R_bg1, R_Wg2, R_bg2,
           nl_W, nl_b, nr_W, nr_b, sf_W, sf_b, ln_g, ln_b, ot_W, ot_b):
    f32 = jnp.float32
    bf16 = jnp.bfloat16

    flat_idx = bond_index.reshape(2 * _E)

    hn2 = _gather_rows(h_node, flat_idx, add_half_offset=False)
    hl = hn2[:_E]
    hr = hn2[_E:]

    gtl = (bond_time * L_Wg1[2 * _D] + L_bg1).astype(bf16)
    gtr = (bond_time * R_Wg1[2 * _D] + R_bg1).astype(bf16)

    def w2(x):
        return x.astype(bf16)

    def b2d(x):
        return x.reshape(1, -1)

    lin_b = b2d(nl_b + nr_b + sf_b)

    weights = [
        w2(L_Wb), w2(L_Wn), w2(L_W1), b2d(L_b1), w2(L_W2), b2d(L_b2),
        w2(L_Wg1[:_D]), w2(L_Wg1[_D:2 * _D]), w2(L_Wg2), b2d(L_bg2),
        w2(R_Wb), w2(R_Wn), w2(R_W1), b2d(R_b1), w2(R_W2), b2d(R_b2),
        w2(R_Wg1[:_D]), w2(R_Wg1[_D:2 * _D]), w2(R_Wg2), b2d(R_bg2),
        w2(nl_W), w2(nr_W), w2(sf_W), lin_b,
    ]

    in_specs = [
        pl.BlockSpec((_BE, _D), lambda i: (i, 0)),
        pl.BlockSpec((_BE, _D), lambda i: (i, 0)),
        pl.BlockSpec((_BE, _D), lambda i: (i + _GRID, 0)),
        pl.BlockSpec((_BE, _DG), lambda i: (i, 0)),
        pl.BlockSpec((_BE, _DG), lambda i: (i, 0)),
    ] + [_full_spec(w.shape) for w in weights]

    msgs, lin = pl.pallas_call(
        _tc1_body,
        grid=(_GRID,),
        in_specs=in_specs,
        out_specs=[
            pl.BlockSpec((2, _BE, _D), lambda i: (0, i, 0)),
            pl.BlockSpec((_BE, _D), lambda i: (i, 0)),
        ],
        out_shape=[
            jax.ShapeDtypeStruct((2, _E, _D), f32),
            jax.ShapeDtypeStruct((_E, _D), bf16),
        ],
        compiler_params=pltpu.CompilerParams(
            dimension_semantics=("parallel",)),
    )(h_bond, hn2, hn2, gtl, gtr, *weights)

    idx_swp = bond_index[::-1].reshape(2 * _E)
    zeros = jnp.zeros((_NP, _D), f32)
    seg = _scatter_sum(msgs, idx_swp, zeros)

    g2 = _gather_rows(seg, flat_idx, add_half_offset=True)

    out = pl.pallas_call(
        _tc2_body,
        grid=(_GRID,),
        in_specs=[
            pl.BlockSpec((_BE, _D), lambda i: (i, 0)),
            pl.BlockSpec((_BE, _D), lambda i: (i + _GRID, 0)),
            pl.BlockSpec((_BE, _D), lambda i: (i, 0)),
            _full_spec((1, _D)),
            _full_spec((1, _D)),
            _full_spec((_D, _D)),
            _full_spec((1, _D)),
        ],
        out_specs=pl.BlockSpec((_BE, _D), lambda i: (i, 0)),
        out_shape=jax.ShapeDtypeStruct((_E, _D), f32),
        compiler_params=pltpu.CompilerParams(
            dimension_semantics=("parallel",)),
    )(g2, g2, lin, b2d(ln_g), b2d(ln_b), w2(ot_W), b2d(ot_b))

    return out


# speedup vs baseline: 2.7378x; 1.0403x over previous
import jax
import jax.numpy as jnp
from jax import lax
from jax.experimental import pallas as pl
from jax.experimental.pallas import tpu as pltpu
from jax.experimental.pallas import tpu_sc as plsc

_N = 10000
_E = 160000
_D = 128
_DI = 256
_DG = 32
_NC, _NS = 2, 16
_NW = _NC * _NS
_CH = 80
_U = 5
_ROWS_W = (2 * _E) // _NW
_NCHUNK = _ROWS_W // _CH
_NSUP = _NCHUNK // _U
_NP = 10240
_RPT = _NP // _NS
_CHS = 40
_NCHS = (_E // _NS) // _CHS
_NSUPS = _NCHS // _U
_EPT = _E // _NS
_BE = 1000
_GRID = _E // _BE


def _sc_mesh():
    return plsc.VectorSubcoreMesh(
        core_axis_name="c", subcore_axis_name="s",
        num_cores=_NC, num_subcores=_NS)


def _gather_rows(table, idx_flat):
    t_rows, w = table.shape
    m = idx_flat.shape[0]
    dt = table.dtype

    def body(table_hbm, idx_hbm, out_hbm, idxv, buf, gsem, ssem):
        c = lax.axis_index("c")
        s = lax.axis_index("s")
        wid = s * _NC + c
        rbase = pl.multiple_of(wid * _ROWS_W, _ROWS_W)
        pltpu.sync_copy(idx_hbm.at[pl.ds(rbase, _ROWS_W)], idxv)

        def sup(t, carry):
            g0 = t * _U
            gets = [
                pltpu.async_copy(
                    table_hbm.at[idxv.at[pl.ds(
                        pl.multiple_of((g0 + b) * _CH, _CH), _CH)]],
                    buf.at[b], gsem)
                for b in range(_U)]
            for d in gets:
                d.wait()
            puts = [
                pltpu.async_copy(
                    buf.at[b],
                    out_hbm.at[pl.ds(rbase + (g0 + b) * _CH, _CH)],
                    ssem)
                for b in range(_U)]
            for d in puts:
                d.wait()
            return carry

        lax.fori_loop(0, _NSUP, sup, 0, unroll=False)

    f = pl.kernel(
        body,
        out_type=jax.ShapeDtypeStruct((m, w), dt),
        mesh=_sc_mesh(),
        scratch_types=[
            pltpu.VMEM((_ROWS_W,), jnp.int32),
            pltpu.VMEM((_U, _CH, w), dt),
            pltpu.SemaphoreType.DMA,
            pltpu.SemaphoreType.DMA,
        ])
    return f(table, idx_flat)


def _scatter_gather(msgs, idx_swp, idx_fwd, zeros):
    def body(msgs_hbm, idxs_hbm, idxf_hbm, zr_hbm, out_hbm, idxv, buf,
             shared, gsem, ssem):
        c = lax.axis_index("c")
        s = lax.axis_index("s")
        zb = pl.multiple_of(s * _RPT, _RPT)
        pltpu.sync_copy(zr_hbm.at[pl.ds(zb, _RPT)], shared.at[pl.ds(zb, _RPT)])
        half = pl.multiple_of(c * _E + s * _EPT, _EPT)
        pltpu.sync_copy(idxs_hbm.at[pl.ds(half, _EPT)], idxv)
        plsc.subcore_barrier()
        ebase = pl.multiple_of(s * _EPT, _EPT)

        def sup(t, carry):
            g0 = t * _U
            gets = [
                pltpu.async_copy(
                    msgs_hbm.at[c, pl.ds(ebase + (g0 + b) * _CHS, _CHS)],
                    buf.at[b], gsem)
                for b in range(_U)]
            for d in gets:
                d.wait()
            puts = [
                pltpu.async_copy(
                    buf.at[b],
                    shared.at[idxv.at[pl.ds(
                        pl.multiple_of((g0 + b) * _CHS, _CHS), _CHS)]],
                    ssem, add=True)
                for b in range(_U)]
            for d in puts:
                d.wait()
            return carry

        lax.fori_loop(0, _NSUPS, sup, 0, unroll=False)
        plsc.subcore_barrier()
        pltpu.sync_copy(idxf_hbm.at[pl.ds(half, _EPT)], idxv)

        def sup2(t, carry):
            g0 = t * _U
            gets = [
                pltpu.async_copy(
                    shared.at[idxv.at[pl.ds(
                        pl.multiple_of((g0 + b) * _CHS, _CHS), _CHS)]],
                    buf.at[b], gsem)
                for b in range(_U)]
            for d in gets:
                d.wait()
            puts = [
                pltpu.async_copy(
                    buf.at[b],
                    out_hbm.at[pl.ds(half + (g0 + b) * _CHS, _CHS)],
                    ssem)
                for b in range(_U)]
            for d in puts:
                d.wait()
            return carry

        lax.fori_loop(0, _NSUPS, sup2, 0, unroll=False)

    f = pl.kernel(
        body,
        out_type=jax.ShapeDtypeStruct((2 * _E, _D), jnp.float32),
        mesh=_sc_mesh(),
        scratch_types=[
            pltpu.VMEM((_EPT,), jnp.int32),
            pltpu.VMEM((_U, _CHS, _D), jnp.float32),
            pltpu.VMEM_SHARED((_NP, _D), jnp.float32),
            pltpu.SemaphoreType.DMA,
            pltpu.SemaphoreType.DMA,
        ])
    return f(msgs, idx_swp, idx_fwd, zeros)


def _tc1_body(hb_ref, hl_ref, hr_ref, gtl_ref, gtr_ref,
              lWb, lWn, lW1, lb1, lW2, lb2, lG1b, lG1n, lWg2, lbg2,
              rWb, rWn, rW1, rb1, rW2, rb2, rG1b, rG1n, rWg2, rbg2,
              nlW, nrW, sfW, lb_ref,
              msgs_ref, lin_ref):
    f32 = jnp.float32
    bf16 = jnp.bfloat16
    hbb = hb_ref[...].astype(bf16)
    hl = hl_ref[...].astype(bf16)
    hr = hr_ref[...].astype(bf16)

    def side(hn, Wb, Wn, W1, b1, W2, b2, G1b, G1n, gt, Wg2, bg2):
        a = jnp.dot(hbb, Wb[...], preferred_element_type=f32)
        bb = jnp.dot(hn, Wn[...], preferred_element_type=f32)
        inter = (a * bb).astype(bf16)
        h1 = jax.nn.relu(
            jnp.dot(inter, W1[...], preferred_element_type=f32) + b1[...])
        inter2 = (jnp.dot(h1.astype(bf16), W2[...],
                          preferred_element_type=f32) + b2[...])
        g = (jnp.dot(hbb, G1b[...], preferred_element_type=f32)
             + jnp.dot(hn, G1n[...], preferred_element_type=f32)
             + gt[...].astype(f32))
        g = jax.nn.relu(g).astype(bf16)
        gate = jnp.dot(g, Wg2[...], preferred_element_type=f32) + bg2[...]
        return inter2 * jax.nn.sigmoid(gate)

    msgs_ref[0] = side(hl, lWb, lWn, lW1, lb1, lW2, lb2,
                       lG1b, lG1n, gtl_ref, lWg2, lbg2)
    msgs_ref[1] = side(hr, rWb, rWn, rW1, rb1, rW2, rb2,
                       rG1b, rG1n, gtr_ref, rWg2, rbg2)
    lin = (jnp.dot(hl, nlW[...], preferred_element_type=f32)
           + jnp.dot(hr, nrW[...], preferred_element_type=f32)
           + jnp.dot(hbb, sfW[...], preferred_element_type=f32)
           + lb_ref[...])
    lin_ref[...] = lin.astype(bf16)


def _tc2_body(gl_ref, gr_ref, lin_ref, lng, lnb, otW, otb, out_ref):
    f32 = jnp.float32
    h = (gl_ref[...].astype(f32) + gr_ref[...].astype(f32)
         + lin_ref[...].astype(f32))
    mu = jnp.mean(h, axis=1, keepdims=True)
    d = h - mu
    var = jnp.mean(d * d, axis=1, keepdims=True)
    hn = d * lax.rsqrt(var + 1e-5) * lng[...] + lnb[...]
    r = jax.nn.relu(hn).astype(jnp.bfloat16)
    out_ref[...] = jnp.dot(r, otW[...], preferred_element_type=f32) + otb[...]


def _full_spec(shape):
    return pl.BlockSpec(shape, lambda i: tuple(0 for _ in shape))


def kernel(h_bond, bond_index, h_node, bond_time,
           L_Wb, L_Wn, L_W1, L_b1, L_W2, L_b2, L_Wg1, L_bg1, L_Wg2, L_bg2,
           R_Wb, R_Wn, R_W1, R_b1, R_W2, R_b2, R_Wg1, R_bg1, R_Wg2, R_bg2,
           nl_W, nl_b, nr_W, nr_b, sf_W, sf_b, ln_g, ln_b, ot_W, ot_b):
    f32 = jnp.float32
    bf16 = jnp.bfloat16

    flat_idx = bond_index.reshape(2 * _E)

    hn2 = _gather_rows(h_node, flat_idx)
    hl = hn2[:_E]
    hr = hn2[_E:]

    gtl = (bond_time * L_Wg1[2 * _D] + L_bg1).astype(bf16)
    gtr = (bond_time * R_Wg1[2 * _D] + R_bg1).astype(bf16)

    def w2(x):
        return x.astype(bf16)

    def b2d(x):
        return x.reshape(1, -1)

    lin_b = b2d(nl_b + nr_b + sf_b)

    weights = [
        w2(L_Wb), w2(L_Wn), w2(L_W1), b2d(L_b1), w2(L_W2), b2d(L_b2),
        w2(L_Wg1[:_D]), w2(L_Wg1[_D:2 * _D]), w2(L_Wg2), b2d(L_bg2),
        w2(R_Wb), w2(R_Wn), w2(R_W1), b2d(R_b1), w2(R_W2), b2d(R_b2),
        w2(R_Wg1[:_D]), w2(R_Wg1[_D:2 * _D]), w2(R_Wg2), b2d(R_bg2),
        w2(nl_W), w2(nr_W), w2(sf_W), lin_b,
    ]

    in_specs = [
        pl.BlockSpec((_BE, _D), lambda i: (i, 0)),
        pl.BlockSpec((_BE, _D), lambda i: (i, 0)),
        pl.BlockSpec((_BE, _D), lambda i: (i + _GRID, 0)),
        pl.BlockSpec((_BE, _DG), lambda i: (i, 0)),
        pl.BlockSpec((_BE, _DG), lambda i: (i, 0)),
    ] + [_full_spec(w.shape) for w in weights]

    msgs, lin = pl.pallas_call(
        _tc1_body,
        grid=(_GRID,),
        in_specs=in_specs,
        out_specs=[
            pl.BlockSpec((2, _BE, _D), lambda i: (0, i, 0)),
            pl.BlockSpec((_BE, _D), lambda i: (i, 0)),
        ],
        out_shape=[
            jax.ShapeDtypeStruct((2, _E, _D), f32),
            jax.ShapeDtypeStruct((_E, _D), bf16),
        ],
        compiler_params=pltpu.CompilerParams(
            dimension_semantics=("parallel",)),
    )(h_bond, hn2, hn2, gtl, gtr, *weights)

    idx_swp = bond_index[::-1].reshape(2 * _E)
    zeros = jnp.zeros((_NP, _D), f32)
    g2 = _scatter_gather(msgs, idx_swp, flat_idx, zeros)

    out = pl.pallas_call(
        _tc2_body,
        grid=(_GRID,),
        in_specs=[
            pl.BlockSpec((_BE, _D), lambda i: (i, 0)),
            pl.BlockSpec((_BE, _D), lambda i: (i + _GRID, 0)),
            pl.BlockSpec((_BE, _D), lambda i: (i, 0)),
            _full_spec((1, _D)),
            _full_spec((1, _D)),
            _full_spec((_D, _D)),
            _full_spec((1, _D)),
        ],
        out_specs=pl.BlockSpec((_BE, _D), lambda i: (i, 0)),
        out_shape=jax.ShapeDtypeStruct((_E, _D), f32),
        compiler_params=pltpu.CompilerParams(
            dimension_semantics=("parallel",)),
    )(g2, g2, lin, b2d(ln_g), b2d(ln_b), w2(ot_W), b2d(ot_b))

    return out
